# baseline (speedup 1.0000x reference)
import functools

import jax
import jax.numpy as jnp
from jax import lax
from jax.experimental import pallas as pl
from jax.experimental.pallas import tpu as pltpu
from jax.experimental.pallas import tpu_sc as plsc

D = 32
NC, NS = 2, 16
NW = NC * NS
CB = 128
RB = D // 8


@functools.partial(jax.jit, static_argnums=(1, 2))
def _gather_rows(args, b, f):
    xt2, weight = args
    nchunk = f * (b // CB)
    per_w = nchunk // NW

    mesh = plsc.VectorSubcoreMesh(core_axis_name="c", subcore_axis_name="s")

    @functools.partial(
        pl.kernel,
        out_type=jax.ShapeDtypeStruct((f, RB, b // CB, 8, CB), jnp.float32),
        mesh=mesh,
        scratch_types=[
            pltpu.VMEM((per_w, CB), jnp.int32),
            pltpu.VMEM((CB, D), jnp.float32),
            pltpu.VMEM((D, CB), jnp.float32),
            pltpu.SemaphoreType.DMA,
        ],
        compiler_params=pltpu.CompilerParams(
            use_tc_tiling_on_sc=False, needs_layout_passes=False
        ),
    )
    def k(xt_hbm, w_hbm, out_hbm, idx_v, rows_v, tr_v, sem):
        wid = lax.axis_index("s") * NC + lax.axis_index("c")
        pltpu.sync_copy(xt_hbm.at[pl.ds(wid * per_w, per_w)], idx_v)
        lane = lax.iota(jnp.int32, 16)

        def chunk(t_local, carry):
            t = wid * per_w + t_local
            fi = t // (b // CB)
            cb = t % (b // CB)
            pltpu.async_copy(w_hbm.at[idx_v.at[t_local]], rows_v, sem).wait()
            for d in range(D):
                for c0 in range(CB // 16):
                    vals = plsc.load_gather(
                        rows_v, [lane + (c0 * 16), jnp.full((16,), d, jnp.int32)]
                    )
                    tr_v[d, pl.ds(c0 * 16, 16)] = vals
            for rb in range(RB):
                pltpu.sync_copy(
                    tr_v.at[pl.ds(rb * 8, 8)], out_hbm.at[fi, rb, cb]
                )
            return carry

        lax.fori_loop(0, per_w, chunk, 0)

    return k(xt2, weight)


def kernel(x, weight):
    b, f = x.shape
    xt2 = x.T.astype(jnp.int32).reshape(f * (b // CB), CB)
    y = _gather_rows((xt2, weight), b, f)
    return y.transpose(2, 4, 0, 1, 3).reshape(b, f, D)
